# Initial kernel scaffold; baseline (speedup 1.0000x reference)
import jax
import jax.numpy as jnp
from jax.experimental import pallas as pl


def kernel(hidden_states, beta_ce, beta_cu, blk_norm_w, blk_wg, blk_wu, blk_wd, tpn_norm_w, tpn_wg, tpn_wu, tpn_wd, cr_w, cr_b):
    raise NotImplementedError("write your pallas kernel here")


# speedup vs baseline: 1.6919x; 1.6919x over previous
import functools

import jax
import jax.numpy as jnp
from jax.experimental import pallas as pl

_EPS = 1e-6
_CAPACITY = 0.5


def _rmsnorm(x, w):
    v = jnp.mean(x * x, axis=-1, keepdims=True)
    return (x * jax.lax.rsqrt(v + _EPS)) * w


def _mlp_block_body(x_ref, nw_ref, wg_ref, wu_ref, wd_ref, out_ref):
    x = x_ref[...]
    xn = _rmsnorm(x, nw_ref[...])
    a = jnp.dot(xn, wg_ref[...], preferred_element_type=jnp.float32)
    b = jnp.dot(xn, wu_ref[...], preferred_element_type=jnp.float32)
    h = (a * jax.nn.sigmoid(a)) * b
    r = jnp.dot(h, wd_ref[...], preferred_element_type=jnp.float32)
    out_ref[...] = x + r


def _mlp_tpn_body(prev_ref, x_ref, proc_ref, nw_ref, wg_ref, wu_ref, wd_ref,
                  betas_ref, g_ref, *, d):
    xp = prev_ref[...]
    xn = _rmsnorm(xp, nw_ref[...])
    a = jnp.dot(xn, wg_ref[...], preferred_element_type=jnp.float32)
    b = jnp.dot(xn, wu_ref[...], preferred_element_type=jnp.float32)
    h = (a * jax.nn.sigmoid(a)) * b
    p = jnp.dot(h, wd_ref[...], preferred_element_type=jnp.float32)
    r = proc_ref[...] - x_ref[...]
    d_st = jnp.sum(r * r, axis=-1, keepdims=True) * (1.0 / d)
    e = r - p
    d_ch = jnp.sum(e * e, axis=-1, keepdims=True) * (1.0 / d)
    beta_ce = betas_ref[0, 0]
    beta_cu = betas_ref[0, 1]
    g_ref[...] = jax.nn.sigmoid(beta_ce * d_st - beta_cu * d_ch)


def _route_body(g_ref, out_ref, *, k):
    g = g_ref[...]
    rows = g.shape[0]
    t = g.shape[1]
    gb = jax.lax.bitcast_convert_type(g, jnp.int32)

    def body1(_, lohi):
        lo, hi = lohi
        mid = lo + (hi - lo + 1) // 2
        cnt = jnp.sum((gb >= mid).astype(jnp.int32), axis=-1, keepdims=True)
        pred = cnt >= k
        return jnp.where(pred, mid, lo), jnp.where(pred, hi, mid - 1)

    lo0 = jnp.zeros((rows, 1), jnp.int32)
    hi0 = jnp.full((rows, 1), 0x3F800000, jnp.int32)
    thresh, _ = jax.lax.fori_loop(0, 31, body1, (lo0, hi0))

    n_gt = jnp.sum((gb > thresh).astype(jnp.int32), axis=-1, keepdims=True)
    need = k - n_gt
    eq = gb == thresh
    iota = jax.lax.broadcasted_iota(jnp.int32, gb.shape, 1)

    def body2(_, lohi):
        lo2, hi2 = lohi
        mid = (lo2 + hi2) // 2
        q = jnp.sum((eq & (iota < mid)).astype(jnp.int32), axis=-1,
                    keepdims=True) >= need
        return jnp.where(q, lo2, mid), jnp.where(q, mid, hi2)

    _, cut = jax.lax.fori_loop(
        0, 12, body2,
        (jnp.zeros((rows, 1), jnp.int32), jnp.full((rows, 1), t, jnp.int32)))

    mask = (gb > thresh) | (eq & (iota < cut))
    out_ref[...] = jnp.where(mask, g, jnp.zeros_like(g))


def _combine_body(x_ref, proc_ref, gated_ref, out_ref):
    x = x_ref[...]
    out_ref[...] = x + gated_ref[...] * (proc_ref[...] - x)


def kernel(hidden_states, beta_ce, beta_cu, blk_norm_w, blk_wg, blk_wu,
           blk_wd, tpn_norm_w, tpn_wg, tpn_wu, tpn_wd, cr_w, cr_b):
    bsz, seq, d = hidden_states.shape
    f = blk_wg.shape[1]
    n = bsz * seq
    m = 256
    grid = (n // m,)
    k = max(1, int(seq * _CAPACITY))

    x = hidden_states.reshape(n, d)
    nw1 = blk_norm_w.reshape(1, d)
    nw2 = tpn_norm_w.reshape(1, d)

    tile_spec = pl.BlockSpec((m, d), lambda i: (i, 0))
    nw_spec = pl.BlockSpec((1, d), lambda i: (0, 0))
    win_spec = pl.BlockSpec((d, f), lambda i: (0, 0))
    wout_spec = pl.BlockSpec((f, d), lambda i: (0, 0))

    processed = pl.pallas_call(
        _mlp_block_body,
        grid=grid,
        in_specs=[tile_spec, nw_spec, win_spec, win_spec, wout_spec],
        out_specs=tile_spec,
        out_shape=jax.ShapeDtypeStruct((n, d), jnp.float32),
    )(x, nw1, blk_wg, blk_wu, blk_wd)

    proc3 = processed.reshape(bsz, seq, d)
    prev = jnp.concatenate(
        [jnp.zeros_like(proc3[:, :1, :]), proc3[:, :-1, :]], axis=1
    ).reshape(n, d)

    betas = jnp.stack([jnp.asarray(beta_ce, jnp.float32),
                       jnp.asarray(beta_cu, jnp.float32)]).reshape(1, 2)

    g = pl.pallas_call(
        functools.partial(_mlp_tpn_body, d=float(d)),
        grid=grid,
        in_specs=[tile_spec, tile_spec, tile_spec, nw_spec, win_spec,
                  win_spec, wout_spec, pl.BlockSpec((1, 2), lambda i: (0, 0))],
        out_specs=pl.BlockSpec((m, 1), lambda i: (i, 0)),
        out_shape=jax.ShapeDtypeStruct((n, 1), jnp.float32),
    )(prev, x, processed, nw2, tpn_wg, tpn_wu, tpn_wd, betas)

    gated = pl.pallas_call(
        functools.partial(_route_body, k=k),
        out_shape=jax.ShapeDtypeStruct((bsz, seq), jnp.float32),
    )(g.reshape(bsz, seq))

    out = pl.pallas_call(
        _combine_body,
        grid=grid,
        in_specs=[tile_spec, tile_spec, pl.BlockSpec((m, 1), lambda i: (i, 0))],
        out_specs=tile_spec,
        out_shape=jax.ShapeDtypeStruct((n, d), jnp.float32),
    )(x, processed, gated.reshape(n, 1))

    return out.reshape(bsz, seq, d)


# speedup vs baseline: 1.8161x; 1.0734x over previous
import functools

import jax
import jax.numpy as jnp
from jax.experimental import pallas as pl
from jax.experimental.pallas import tpu as pltpu

_EPS = 1e-6
_CAPACITY = 0.5


def _rmsnorm(x, w):
    v = jnp.mean(x * x, axis=-1, keepdims=True)
    return (x * jax.lax.rsqrt(v + _EPS)) * w


def _swiglu(xn, wg_ref, wu_ref, wd_ref):
    a = jnp.dot(xn, wg_ref[...], preferred_element_type=jnp.float32)
    b = jnp.dot(xn, wu_ref[...], preferred_element_type=jnp.float32)
    h = (a * jax.nn.sigmoid(a)) * b
    return jnp.dot(h, wd_ref[...], preferred_element_type=jnp.float32)


def _mlp_block_body(x_ref, nw_ref, wg_ref, wu_ref, wd_ref, out_ref):
    x = x_ref[...]
    out_ref[...] = x + _swiglu(_rmsnorm(x, nw_ref[...]), wg_ref, wu_ref, wd_ref)


def _mlp_tpn_body(pprev_ref, x_ref, proc_ref, nw_ref, wg_ref, wu_ref, wd_ref,
                  betas_ref, g_ref, *, d, tiles_per_seq):
    m = x_ref.shape[0]
    i = pl.program_id(0)
    proc = proc_ref[...]
    prev = jnp.concatenate([pprev_ref[m - 1:m, :], proc[:m - 1, :]], axis=0)
    seq_start = (i % tiles_per_seq) == 0
    row0 = jax.lax.broadcasted_iota(jnp.int32, (m, 1), 0) == 0
    prev = jnp.where(row0 & seq_start, 0.0, prev)
    p = _swiglu(_rmsnorm(prev, nw_ref[...]), wg_ref, wu_ref, wd_ref)
    r = proc - x_ref[...]
    d_st = jnp.sum(r * r, axis=-1, keepdims=True) * (1.0 / d)
    e = r - p
    d_ch = jnp.sum(e * e, axis=-1, keepdims=True) * (1.0 / d)
    g_ref[...] = jax.nn.sigmoid(betas_ref[0, 0] * d_st - betas_ref[0, 1] * d_ch)


def _route_combine_body(gfull_ref, x_ref, proc_ref, g_ref, out_ref, th_ref,
                        *, k, tiles_per_seq):
    i = pl.program_id(0)
    m = x_ref.shape[0]

    @pl.when(i == 0)
    def _():
        g = gfull_ref[...]
        rows, t = g.shape
        gb = jax.lax.bitcast_convert_type(g, jnp.int32)

        def body1(_, lohi):
            lo, hi = lohi
            mid = lo + (hi - lo + 1) // 2
            cnt = jnp.sum((gb >= mid).astype(jnp.int32), axis=-1,
                          keepdims=True)
            pred = cnt >= k
            return jnp.where(pred, mid, lo), jnp.where(pred, hi, mid - 1)

        lo0 = jnp.zeros((rows, 1), jnp.int32)
        hi0 = jnp.full((rows, 1), 0x3F800000, jnp.int32)
        thresh, _ = jax.lax.fori_loop(0, 31, body1, (lo0, hi0))

        n_gt = jnp.sum((gb > thresh).astype(jnp.int32), axis=-1, keepdims=True)
        need = k - n_gt
        eq = gb == thresh
        iota = jax.lax.broadcasted_iota(jnp.int32, gb.shape, 1)

        def body2(_, lohi):
            lo2, hi2 = lohi
            mid = (lo2 + hi2) // 2
            q = jnp.sum((eq & (iota < mid)).astype(jnp.int32), axis=-1,
                        keepdims=True) >= need
            return jnp.where(q, lo2, mid), jnp.where(q, mid, hi2)

        _, cut = jax.lax.fori_loop(
            0, 12, body2,
            (jnp.zeros((rows, 1), jnp.int32), jnp.full((rows, 1), t,
                                                       jnp.int32)))
        th_ref[0:rows, :] = jnp.broadcast_to(thresh, (rows, 128))
        th_ref[rows:2 * rows, :] = jnp.broadcast_to(cut, (rows, 128))

    b = i // tiles_per_seq
    t0 = (i % tiles_per_seq) * m
    rows = gfull_ref.shape[0]
    thr_b = th_ref[pl.ds(b, 1), 0:1]
    cut_b = th_ref[pl.ds(b + rows, 1), 0:1]
    g = g_ref[...]
    gb = jax.lax.bitcast_convert_type(g, jnp.int32)
    tloc = jax.lax.broadcasted_iota(jnp.int32, (m, 1), 0) + t0
    mask = (gb > thr_b) | ((gb == thr_b) & (tloc < cut_b))
    x = x_ref[...]
    gated = jnp.where(mask, g, jnp.zeros_like(g))
    out_ref[...] = x + gated * (proc_ref[...] - x)


def kernel(hidden_states, beta_ce, beta_cu, blk_norm_w, blk_wg, blk_wu,
           blk_wd, tpn_norm_w, tpn_wg, tpn_wu, tpn_wd, cr_w, cr_b):
    bsz, seq, d = hidden_states.shape
    f = blk_wg.shape[1]
    n = bsz * seq
    m = min(256, seq)
    nt = n // m
    tiles_per_seq = seq // m
    grid = (nt,)
    k = max(1, int(seq * _CAPACITY))

    x = hidden_states.reshape(n, d)
    nw1 = blk_norm_w.reshape(1, d)
    nw2 = tpn_norm_w.reshape(1, d)

    tile_spec = pl.BlockSpec((m, d), lambda i: (i, 0))
    prev_spec = pl.BlockSpec((m, d), lambda i: (jnp.maximum(i - 1, 0), 0))
    nw_spec = pl.BlockSpec((1, d), lambda i: (0, 0))
    win_spec = pl.BlockSpec((d, f), lambda i: (0, 0))
    wout_spec = pl.BlockSpec((f, d), lambda i: (0, 0))
    gtile_spec = pl.BlockSpec((m, 1), lambda i: (i, 0))

    processed = pl.pallas_call(
        _mlp_block_body,
        grid=grid,
        in_specs=[tile_spec, nw_spec, win_spec, win_spec, wout_spec],
        out_specs=tile_spec,
        out_shape=jax.ShapeDtypeStruct((n, d), jnp.float32),
    )(x, nw1, blk_wg, blk_wu, blk_wd)

    betas = jnp.stack([jnp.asarray(beta_ce, jnp.float32),
                       jnp.asarray(beta_cu, jnp.float32)]).reshape(1, 2)

    g = pl.pallas_call(
        functools.partial(_mlp_tpn_body, d=float(d),
                          tiles_per_seq=tiles_per_seq),
        grid=grid,
        in_specs=[prev_spec, tile_spec, tile_spec, nw_spec, win_spec,
                  win_spec, wout_spec, pl.BlockSpec((1, 2), lambda i: (0, 0))],
        out_specs=gtile_spec,
        out_shape=jax.ShapeDtypeStruct((n, 1), jnp.float32),
    )(processed, x, processed, nw2, tpn_wg, tpn_wu, tpn_wd, betas)

    out = pl.pallas_call(
        functools.partial(_route_combine_body, k=k,
                          tiles_per_seq=tiles_per_seq),
        grid=grid,
        in_specs=[pl.BlockSpec((bsz, seq), lambda i: (0, 0)), tile_spec,
                  tile_spec, gtile_spec],
        out_specs=tile_spec,
        out_shape=jax.ShapeDtypeStruct((n, d), jnp.float32),
        scratch_shapes=[pltpu.VMEM((2 * bsz, 128), jnp.int32)],
    )(g.reshape(bsz, seq), x, processed, g)

    return out.reshape(bsz, seq, d)


# speedup vs baseline: 1.8490x; 1.0181x over previous
import functools

import jax
import jax.numpy as jnp
from jax.experimental import pallas as pl
from jax.experimental.pallas import tpu as pltpu

_EPS = 1e-6
_CAPACITY = 0.5


def _rmsnorm(x, w):
    v = jnp.mean(x * x, axis=-1, keepdims=True)
    return (x * jax.lax.rsqrt(v + _EPS)) * w


def _swiglu(xn, wg_ref, wu_ref, wd_ref):
    a = jnp.dot(xn, wg_ref[...], preferred_element_type=jnp.float32)
    b = jnp.dot(xn, wu_ref[...], preferred_element_type=jnp.float32)
    h = (a * jax.nn.sigmoid(a)) * b
    return jnp.dot(h, wd_ref[...], preferred_element_type=jnp.float32)


def _mlp_block_body(x_ref, nw_ref, wg_ref, wu_ref, wd_ref, out_ref):
    x = x_ref[...]
    out_ref[...] = x + _swiglu(_rmsnorm(x, nw_ref[...]), wg_ref, wu_ref, wd_ref)


def _mlp_tpn_body(pprev_ref, x_ref, proc_ref, nw_ref, wg_ref, wu_ref, wd_ref,
                  betas_ref, g_ref, *, d, tiles_per_seq):
    m = x_ref.shape[0]
    i = pl.program_id(0)
    proc = proc_ref[...]
    prev = jnp.concatenate([pprev_ref[m - 1:m, :], proc[:m - 1, :]], axis=0)
    seq_start = (i % tiles_per_seq) == 0
    row0 = jax.lax.broadcasted_iota(jnp.int32, (m, 1), 0) == 0
    prev = jnp.where(row0 & seq_start, 0.0, prev)
    p = _swiglu(_rmsnorm(prev, nw_ref[...]), wg_ref, wu_ref, wd_ref)
    r = proc - x_ref[...]
    d_st = jnp.sum(r * r, axis=-1, keepdims=True) * (1.0 / d)
    e = r - p
    d_ch = jnp.sum(e * e, axis=-1, keepdims=True) * (1.0 / d)
    g_ref[...] = jax.nn.sigmoid(betas_ref[0, 0] * d_st - betas_ref[0, 1] * d_ch)


def _route_combine_body(gfull_ref, x_ref, proc_ref, g_ref, out_ref, th_ref,
                        *, k, tiles_per_seq):
    i = pl.program_id(0)
    m = x_ref.shape[0]

    @pl.when(i == 0)
    def _():
        g = gfull_ref[...]
        rows, t = g.shape
        gb = jax.lax.bitcast_convert_type(g, jnp.int32)

        def body1(_, lohi):
            lo, hi = lohi
            mid = lo + (hi - lo + 1) // 2
            cnt = jnp.sum((gb >= mid).astype(jnp.int32), axis=-1,
                          keepdims=True)
            pred = cnt >= k
            return jnp.where(pred, mid, lo), jnp.where(pred, hi, mid - 1)

        lo0 = jnp.zeros((rows, 1), jnp.int32)
        hi0 = jnp.full((rows, 1), 0x3F800000, jnp.int32)
        thresh, _ = jax.lax.fori_loop(0, 31, body1, (lo0, hi0))

        n_gt = jnp.sum((gb > thresh).astype(jnp.int32), axis=-1, keepdims=True)
        need = k - n_gt
        eq = gb == thresh
        iota = jax.lax.broadcasted_iota(jnp.int32, gb.shape, 1)

        def body2(_, lohi):
            lo2, hi2 = lohi
            mid = (lo2 + hi2) // 2
            q = jnp.sum((eq & (iota < mid)).astype(jnp.int32), axis=-1,
                        keepdims=True) >= need
            return jnp.where(q, lo2, mid), jnp.where(q, mid, hi2)

        _, cut = jax.lax.fori_loop(
            0, 12, body2,
            (jnp.zeros((rows, 1), jnp.int32), jnp.full((rows, 1), t,
                                                       jnp.int32)))
        th_ref[0:rows, :] = jnp.broadcast_to(thresh, (rows, 128))
        th_ref[rows:2 * rows, :] = jnp.broadcast_to(cut, (rows, 128))

    b = i // tiles_per_seq
    t0 = (i % tiles_per_seq) * m
    rows = gfull_ref.shape[0]
    thr_b = th_ref[pl.ds(b, 1), 0:1]
    cut_b = th_ref[pl.ds(b + rows, 1), 0:1]
    g = g_ref[...]
    gb = jax.lax.bitcast_convert_type(g, jnp.int32)
    tloc = jax.lax.broadcasted_iota(jnp.int32, (m, 1), 0) + t0
    mask = (gb > thr_b) | ((gb == thr_b) & (tloc < cut_b))
    x = x_ref[...]
    gated = jnp.where(mask, g, jnp.zeros_like(g))
    out_ref[...] = x + gated * (proc_ref[...] - x)


def kernel(hidden_states, beta_ce, beta_cu, blk_norm_w, blk_wg, blk_wu,
           blk_wd, tpn_norm_w, tpn_wg, tpn_wu, tpn_wd, cr_w, cr_b):
    bsz, seq, d = hidden_states.shape
    f = blk_wg.shape[1]
    n = bsz * seq
    m = min(512, seq)
    mc = min(256, seq)
    tiles_per_seq = seq // m
    cparams = pltpu.CompilerParams(vmem_limit_bytes=110 * 1024 * 1024)
    k = max(1, int(seq * _CAPACITY))

    x = hidden_states.reshape(n, d)
    nw1 = blk_norm_w.reshape(1, d)
    nw2 = tpn_norm_w.reshape(1, d)

    tile_spec = pl.BlockSpec((m, d), lambda i: (i, 0))
    prev_spec = pl.BlockSpec((m, d), lambda i: (jnp.maximum(i - 1, 0), 0))
    nw_spec = pl.BlockSpec((1, d), lambda i: (0, 0))
    win_spec = pl.BlockSpec((d, f), lambda i: (0, 0))
    wout_spec = pl.BlockSpec((f, d), lambda i: (0, 0))

    processed = pl.pallas_call(
        _mlp_block_body,
        grid=(n // m,),
        in_specs=[tile_spec, nw_spec, win_spec, win_spec, wout_spec],
        out_specs=tile_spec,
        out_shape=jax.ShapeDtypeStruct((n, d), jnp.float32),
        compiler_params=cparams,
    )(x, nw1, blk_wg, blk_wu, blk_wd)

    betas = jnp.stack([jnp.asarray(beta_ce, jnp.float32),
                       jnp.asarray(beta_cu, jnp.float32)]).reshape(1, 2)

    g = pl.pallas_call(
        functools.partial(_mlp_tpn_body, d=float(d),
                          tiles_per_seq=tiles_per_seq),
        grid=(n // m,),
        in_specs=[prev_spec, tile_spec, tile_spec, nw_spec, win_spec,
                  win_spec, wout_spec, pl.BlockSpec((1, 2), lambda i: (0, 0))],
        out_specs=pl.BlockSpec((m, 1), lambda i: (i, 0)),
        out_shape=jax.ShapeDtypeStruct((n, 1), jnp.float32),
        compiler_params=cparams,
    )(processed, x, processed, nw2, tpn_wg, tpn_wu, tpn_wd, betas)

    ctile_spec = pl.BlockSpec((mc, d), lambda i: (i, 0))
    out = pl.pallas_call(
        functools.partial(_route_combine_body, k=k,
                          tiles_per_seq=seq // mc),
        grid=(n // mc,),
        in_specs=[pl.BlockSpec((bsz, seq), lambda i: (0, 0)), ctile_spec,
                  ctile_spec, pl.BlockSpec((mc, 1), lambda i: (i, 0))],
        out_specs=ctile_spec,
        out_shape=jax.ShapeDtypeStruct((n, d), jnp.float32),
        scratch_shapes=[pltpu.VMEM((2 * bsz, 128), jnp.int32)],
    )(g.reshape(bsz, seq), x, processed, g)

    return out.reshape(bsz, seq, d)
